# baseline (speedup 1.0000x reference)
import functools

import jax
import jax.numpy as jnp
from jax import lax
from jax.experimental import pallas as pl
from jax.experimental.pallas import tpu as pltpu
from jax.experimental.pallas import tpu_sc as plsc

NC = 2
NS = 16
NW = NC * NS
LANE = 128


def _transpose_body(nblk, d, tT_hbm, tlin_hbm, inb, outb, isem, osem):
    wid = lax.axis_index("s") * NC + lax.axis_index("c")
    iota = lax.iota(jnp.int32, 16)
    nphase = (nblk + NW - 1) // NW

    def blk(t):
        c = wid + NW * t
        return jnp.where(c < nblk, c, wid)

    def in_copy(t, p):
        return pltpu.make_async_copy(
            tT_hbm.at[:, pl.ds(blk(t) * LANE, LANE)], inb.at[p], isem.at[p])

    def out_copy(t, p):
        return pltpu.make_async_copy(
            outb.at[p], tlin_hbm.at[pl.ds(blk(t) * LANE, LANE)], osem.at[p])

    def transpose(p):
        def body(i, _):
            for u in range(4):
                dd = i * 4 + u
                col = jnp.full((16,), dd, jnp.int32)
                for g in range(8):
                    v = inb[p, dd, pl.ds(16 * g, 16)]
                    plsc.store_scatter(outb.at[p], [iota + 16 * g, col], v)
            return _
        lax.fori_loop(0, d // 4, body, 0)

    in_copy(0, 0).start()
    in_copy(1, 1).start()

    def phase(t, p):
        in_copy(t, p).wait()

        @pl.when(t >= 2)
        def _():
            out_copy(t - 2, p).wait()

        transpose(p)
        out_copy(t, p).start()

        @pl.when(t + 2 < nphase)
        def _():
            in_copy(t + 2, p).start()

    def pair(k, _):
        phase(2 * k, 0)
        phase(2 * k + 1, 1)
        return _

    lax.fori_loop(0, nphase // 2, pair, 0)
    if nphase % 2:
        phase(nphase - 1, 0)
        out_copy(nphase - 2, 1).wait()
        out_copy(nphase - 1, 0).wait()
    else:
        out_copy(nphase - 2, 0).wait()
        out_copy(nphase - 1, 1).wait()


def _gather_body(hist, d, jb, idxT_hbm, tlin_hbm, res3_hbm,
                 idxv, rows, outb, xsem, rsem, osem):
    wid = lax.axis_index("s") * NC + lax.axis_index("c")
    iota = lax.iota(jnp.int32, 16)
    b0 = wid * jb * LANE
    nt = hist * jb

    def stage_idx(h):
        return pltpu.make_async_copy(
            idxT_hbm.at[h, pl.ds(b0, jb * LANE)],
            idxv.at[pl.ds(h * jb * LANE, jb * LANE)], xsem)

    def ld(h, _):
        stage_idx(h).start()
        return _

    def wt(h, _):
        stage_idx(h).wait()
        return _

    lax.fori_loop(0, hist, ld, 0)
    lax.fori_loop(0, hist, wt, 0)

    def gather(t, p):
        return pltpu.make_async_copy(
            tlin_hbm.at[idxv.at[pl.ds(t * LANE, LANE)]], rows.at[p], rsem.at[p])

    def out_copy(t, p):
        h = t // jb
        col = b0 + (t % jb) * LANE
        return pltpu.make_async_copy(
            outb.at[p], res3_hbm.at[h, :, pl.ds(col, LANE)], osem.at[p])

    def transpose(p):
        def body(i, _):
            for u in range(4):
                l = i * 4 + u
                col = jnp.full((16,), l, jnp.int32)
                for g in range(d // 16):
                    v = rows[p, l, pl.ds(16 * g, 16)]
                    plsc.store_scatter(outb.at[p], [iota + 16 * g, col], v)
            return _
        lax.fori_loop(0, LANE // 4, body, 0)

    gather(0, 0).start()
    gather(1, 1).start()

    def phase(t, p):
        gather(t, p).wait()

        @pl.when(t >= 2)
        def _():
            out_copy(t - 2, p).wait()

        transpose(p)
        out_copy(t, p).start()

        @pl.when(t + 2 < nt)
        def _():
            gather(t + 2, p).start()

    def pair(k, _):
        phase(2 * k, 0)
        phase(2 * k + 1, 1)
        return _

    lax.fori_loop(0, nt // 2, pair, 0)
    out_copy(nt - 2, 0).wait()
    out_copy(nt - 1, 1).wait()


def kernel(x, table):
    batch, hist = x.shape
    vocab, d = table.shape
    vp = ((vocab + LANE - 1) // LANE) * LANE
    nblk = vp // LANE
    jb = batch // (LANE * NW)
    assert jb * LANE * NW == batch and (hist * jb) % 2 == 0

    tableT = table.T
    idxT = x.T.astype(jnp.int32)
    mesh = plsc.VectorSubcoreMesh(core_axis_name="c", subcore_axis_name="s",
                                  num_cores=NC, num_subcores=NS)
    cp = pltpu.CompilerParams(use_tc_tiling_on_sc=True,
                              needs_layout_passes=False)

    tlin = pl.kernel(
        functools.partial(_transpose_body, nblk, d),
        out_type=jax.ShapeDtypeStruct((vp, LANE), jnp.float32),
        mesh=mesh, compiler_params=cp,
        scratch_types=[
            pltpu.VMEM((2, d, LANE), jnp.float32),
            pltpu.VMEM((2, LANE, LANE), jnp.float32),
            pltpu.SemaphoreType.DMA((2,)),
            pltpu.SemaphoreType.DMA((2,)),
        ],
    )(tableT)

    res3 = pl.kernel(
        functools.partial(_gather_body, hist, d, jb),
        out_type=jax.ShapeDtypeStruct((hist, d, batch), jnp.float32),
        mesh=mesh, compiler_params=cp,
        scratch_types=[
            pltpu.VMEM((hist * jb * LANE,), jnp.int32),
            pltpu.VMEM((2, LANE, LANE), jnp.float32),
            pltpu.VMEM((2, d, LANE), jnp.float32),
            pltpu.SemaphoreType.DMA,
            pltpu.SemaphoreType.DMA((2,)),
            pltpu.SemaphoreType.DMA((2,)),
        ],
    )(idxT, tlin)

    return jnp.transpose(res3, (2, 0, 1))


# speedup vs baseline: 1.7540x; 1.7540x over previous
import functools

import jax
import jax.numpy as jnp
from jax import lax
from jax.experimental import pallas as pl
from jax.experimental.pallas import tpu as pltpu
from jax.experimental.pallas import tpu_sc as plsc

NC = 2
NS = 16
NW = NC * NS

CHUNK = 128
NBUF = 4


def _emb_body(n_chunks, d, x_hbm, table_hbm, out_hbm, idx_v, rows_v, gsem, osem):
    wid = lax.axis_index("s") * NC + lax.axis_index("c")
    row0 = wid * n_chunks

    pltpu.sync_copy(x_hbm.at[pl.ds(row0, n_chunks)], idx_v)

    def gather(j, b):
        return pltpu.make_async_copy(
            table_hbm.at[idx_v.at[j]], rows_v.at[b], gsem.at[b])

    def writeback(j, b):
        return pltpu.make_async_copy(
            rows_v.at[b], out_hbm.at[pl.ds((row0 + j) * CHUNK, CHUNK)],
            osem.at[b])

    for b in range(NBUF):
        gather(b, b).start()

    def group_body(g, c):
        base = g * NBUF
        for b in range(NBUF):
            gather(base + b, b).wait()
            writeback(base + b, b).start()
        for b in range(NBUF):
            writeback(base + b, b).wait()
            gather(base + NBUF + b, b).start()
        return c

    ngroups = n_chunks // NBUF
    lax.fori_loop(0, ngroups - 1, group_body, 0)

    base = (ngroups - 1) * NBUF
    for b in range(NBUF):
        gather(base + b, b).wait()
        writeback(base + b, b).start()
    for b in range(NBUF):
        writeback(base + b, b).wait()


def kernel(x, table):
    batch, hist = x.shape
    vocab, d = table.shape
    n = batch * hist
    n_rows = n // CHUNK
    n_chunks = n_rows // NW
    assert n_rows * CHUNK == n and n_chunks * NW == n_rows
    assert n_chunks % NBUF == 0

    xf = x.reshape(n_rows, CHUNK).astype(jnp.int32)
    mesh = plsc.VectorSubcoreMesh(core_axis_name="c", subcore_axis_name="s")
    out = pl.kernel(
        functools.partial(_emb_body, n_chunks, d),
        out_type=jax.ShapeDtypeStruct((n, d), table.dtype),
        mesh=mesh,
        compiler_params=pltpu.CompilerParams(use_tc_tiling_on_sc=False),
        scratch_types=[
            pltpu.VMEM((n_chunks, CHUNK), jnp.int32),
            pltpu.VMEM((NBUF, CHUNK, d), jnp.float32),
            pltpu.SemaphoreType.DMA((NBUF,)),
            pltpu.SemaphoreType.DMA((NBUF,)),
        ],
    )(xf, table)
    return out.reshape(batch, hist, d)


# speedup vs baseline: 1.7585x; 1.0025x over previous
import functools

import jax
import jax.numpy as jnp
from jax import lax
from jax.experimental import pallas as pl
from jax.experimental.pallas import tpu as pltpu
from jax.experimental.pallas import tpu_sc as plsc

NC = 2
NS = 16
NW = NC * NS

CHUNK = 128
NBUF = 8


def _emb_body(n_chunks, d, x_hbm, table_hbm, out_hbm, idx_v, rows_v, gsem, osem):
    wid = lax.axis_index("s") * NC + lax.axis_index("c")
    row0 = wid * n_chunks

    pltpu.sync_copy(x_hbm.at[pl.ds(row0, n_chunks)], idx_v)

    def gather(j, b):
        return pltpu.make_async_copy(
            table_hbm.at[idx_v.at[j]], rows_v.at[b], gsem.at[b])

    def writeback(j, b):
        return pltpu.make_async_copy(
            rows_v.at[b], out_hbm.at[pl.ds((row0 + j) * CHUNK, CHUNK)],
            osem.at[b])

    for b in range(NBUF):
        gather(b, b).start()

    def group_body(g, c):
        base = g * NBUF
        for b in range(NBUF):
            gather(base + b, b).wait()
            writeback(base + b, b).start()
        for b in range(NBUF):
            writeback(base + b, b).wait()
            gather(base + NBUF + b, b).start()
        return c

    ngroups = n_chunks // NBUF
    lax.fori_loop(0, ngroups - 1, group_body, 0)

    base = (ngroups - 1) * NBUF
    for b in range(NBUF):
        gather(base + b, b).wait()
        writeback(base + b, b).start()
    for b in range(NBUF):
        writeback(base + b, b).wait()


def kernel(x, table):
    batch, hist = x.shape
    vocab, d = table.shape
    n = batch * hist
    n_rows = n // CHUNK
    n_chunks = n_rows // NW
    assert n_rows * CHUNK == n and n_chunks * NW == n_rows
    assert n_chunks % NBUF == 0

    xf = x.reshape(n_rows, CHUNK).astype(jnp.int32)
    mesh = plsc.VectorSubcoreMesh(core_axis_name="c", subcore_axis_name="s")
    out = pl.kernel(
        functools.partial(_emb_body, n_chunks, d),
        out_type=jax.ShapeDtypeStruct((n, d), table.dtype),
        mesh=mesh,
        compiler_params=pltpu.CompilerParams(use_tc_tiling_on_sc=False),
        scratch_types=[
            pltpu.VMEM((n_chunks, CHUNK), jnp.int32),
            pltpu.VMEM((NBUF, CHUNK, d), jnp.float32),
            pltpu.SemaphoreType.DMA((NBUF,)),
            pltpu.SemaphoreType.DMA((NBUF,)),
        ],
    )(xf, table)
    return out.reshape(batch, hist, d)
